# baseline (speedup 1.0000x reference)
import functools

import numpy as np

import jax
import jax.numpy as jnp
from jax import lax
from jax.experimental import pallas as pl
from jax.experimental.pallas import tpu as pltpu
from jax.experimental.pallas import tpu_sc as plsc

B = 8
N = 4096
CIN = 6
KG = 512
KN = 64
R2 = np.float32(0.8 * 0.8)
NROWS = B * N
ZROW = NROWS
NPAD = NROWS + KG
NW = 32
QPW = (B * KG) // NW


def _fps_body(pts_ref, out_ref):
    lane = lax.broadcasted_iota(jnp.int32, (B, N), 1)

    def step(k, carry):
        min_d, last = carry
        m = lane == last
        qs = []
        for c in range(CIN):
            pc = pts_ref[c]
            qs.append(jnp.sum(jnp.where(m, pc, 0.0), axis=1, keepdims=True))
        row = jnp.concatenate(qs, axis=1)
        out_ref[pl.ds(k, 1)] = row[None]
        e = []
        for c in range(CIN):
            t = pts_ref[c] - qs[c]
            e.append(t * t)
        d = ((e[0] + e[4]) + e[2]) + ((e[1] + e[5]) + e[3])
        min_d = jnp.minimum(min_d, d)
        mx = jnp.max(min_d, axis=1, keepdims=True)
        nxt = jnp.min(jnp.where(min_d == mx, lane, jnp.int32(N)),
                      axis=1, keepdims=True)
        return min_d, nxt

    init = (jnp.full((B, N), jnp.inf, jnp.float32),
            jnp.zeros((B, 1), jnp.int32))
    lax.fori_loop(0, KG, step, init)


_FPS = pl.pallas_call(
    _fps_body,
    out_shape=jax.ShapeDtypeStruct((KG, B, CIN), jnp.float32),
)


_MLP_BLK = NPAD // 8


def _mlp_body(x_ref, w0_ref, b0_ref, w1_ref, b1_ref, w2_ref, b2_ref, o_ref):
    h = jnp.dot(x_ref[...], w0_ref[...], preferred_element_type=jnp.float32)
    h = jnp.maximum(h + b0_ref[...], 0.0)
    h = jnp.dot(h, w1_ref[...], preferred_element_type=jnp.float32)
    h = jnp.maximum(h + b1_ref[...], 0.0)
    h = jnp.dot(h, w2_ref[...], preferred_element_type=jnp.float32)
    o_ref[...] = jnp.maximum(h + b2_ref[...], 0.0)


_MLP = pl.pallas_call(
    _mlp_body,
    grid=(8,),
    in_specs=[
        pl.BlockSpec((_MLP_BLK, 8), lambda i: (i, 0)),
        pl.BlockSpec((8, 64), lambda i: (0, 0)),
        pl.BlockSpec((1, 64), lambda i: (0, 0)),
        pl.BlockSpec((64, 64), lambda i: (0, 0)),
        pl.BlockSpec((1, 64), lambda i: (0, 0)),
        pl.BlockSpec((64, 128), lambda i: (0, 0)),
        pl.BlockSpec((1, 128), lambda i: (0, 0)),
    ],
    out_specs=pl.BlockSpec((_MLP_BLK, 128), lambda i: (i, 0)),
    out_shape=jax.ShapeDtypeStruct((NPAD, 128), jnp.float32),
)


def _dist_body(p1_ref, pts_ref, o_ref):
    q = p1_ref[0]
    kt = pts_ref[0]
    n1 = jnp.sum(q * q, axis=1, keepdims=True)
    n2 = jnp.sum(kt * kt, axis=0, keepdims=True)
    dt = jnp.dot(q, kt, preferred_element_type=jnp.float32)
    o_ref[0] = (n1 + n2) - 2.0 * dt


_DIST = pl.pallas_call(
    _dist_body,
    grid=(8,),
    in_specs=[
        pl.BlockSpec((1, KG, CIN), lambda i: (i, 0, 0)),
        pl.BlockSpec((1, CIN, N), lambda i: (i, 0, 0)),
    ],
    out_specs=pl.BlockSpec((1, KG, N), lambda i: (i, 0, 0)),
    out_shape=jax.ShapeDtypeStruct((B, KG, N), jnp.float32),
)


_GV = 8
_NG = N // (16 * _GV)


def _pool_body(d2_hbm, h2_hbm, out_hbm,
               d2row_v, idxb_v, idx_v, rows_v, out_v,
               dsem_a, dsem_b, gsem_a, gsem_b):
    c = lax.axis_index("c")
    s = lax.axis_index("s")
    w = s * 2 + c
    b = w // 4
    qbase = w * QPW
    boff_f = (b * N).astype(jnp.float32)

    lane0 = lax.iota(jnp.int32, 16) == 0
    iota_f = lax.iota(jnp.int32, 16).astype(jnp.float32)
    zrow_f = np.float32(float(ZROW))
    zfill = jnp.full((16,), zrow_f, jnp.float32)

    def scan(dbuf):
        for u in range(6):
            idxb_v[pl.ds(u * 16, 16)] = zfill

        def gbody(g, cnt):
            def active():
                cc = cnt
                for v in range(_GV):
                    kv16 = (g * _GV + v) * 16
                    dd = d2row_v[dbuf, pl.ds(kv16, 16)]
                    mf = jnp.where(dd < R2, 1.0, 0.0).astype(jnp.float32)
                    vals_f = boff_f + kv16.astype(jnp.float32) + iota_f
                    for l in range(16):
                        keep_f = mf[l]
                        dst = jnp.where(
                            jnp.logical_and(keep_f > np.float32(0.5),
                                            cc < KN),
                            cc, jnp.int32(80))
                        idxb_v[pl.ds(dst, 16)] = jnp.where(
                            lane0, vals_f[l], zrow_f)
                        cc = cc + keep_f.astype(jnp.int32)
                return cc

            return lax.cond(cnt < KN, active, lambda: cnt)

        lax.fori_loop(0, _NG, gbody, jnp.int32(0))

    def conv_idx(p):
        for u in range(4):
            idx_v[p, pl.ds(u * 16, 16)] = (
                idxb_v[pl.ds(u * 16, 16)].astype(jnp.int32))

    def maxq(p, qi):
        def rbody(r, accs):
            return tuple(
                jnp.maximum(accs[v], rows_v[p, r, pl.ds(v * 16, 16)])
                for v in range(8))

        accs = tuple(jnp.zeros((16,), jnp.float32) for _ in range(8))
        accs = lax.fori_loop(0, KN, rbody, accs)
        for v in range(8):
            out_v[qi, pl.ds(v * 16, 16)] = accs[v]

    pltpu.async_copy(d2_hbm.at[qbase], d2row_v.at[0], dsem_a)

    def pair(j, carry):
        e = 2 * j
        o = e + 1
        pltpu.make_async_copy(
            d2_hbm.at[qbase + e], d2row_v.at[0], dsem_a).wait()
        pltpu.async_copy(d2_hbm.at[qbase + o], d2row_v.at[1], dsem_b)
        scan(0)
        conv_idx(0)
        pltpu.async_copy(h2_hbm.at[idx_v.at[0]], rows_v.at[0], gsem_a)

        @pl.when(j > 0)
        def _():
            pltpu.make_async_copy(
                h2_hbm.at[idx_v.at[1]], rows_v.at[1], gsem_b).wait()
            maxq(1, e - 1)

        pltpu.make_async_copy(
            d2_hbm.at[qbase + o], d2row_v.at[1], dsem_b).wait()

        @pl.when(j < QPW // 2 - 1)
        def _():
            pltpu.async_copy(
                d2_hbm.at[qbase + e + 2], d2row_v.at[0], dsem_a)

        scan(1)
        conv_idx(1)
        pltpu.async_copy(h2_hbm.at[idx_v.at[1]], rows_v.at[1], gsem_b)
        pltpu.make_async_copy(
            h2_hbm.at[idx_v.at[0]], rows_v.at[0], gsem_a).wait()
        maxq(0, e)
        return carry

    lax.fori_loop(0, QPW // 2, pair, 0)
    pltpu.make_async_copy(
        h2_hbm.at[idx_v.at[1]], rows_v.at[1], gsem_b).wait()
    maxq(1, QPW - 1)
    pltpu.sync_copy(out_v, out_hbm.at[pl.ds(qbase, QPW)])


@functools.cache
def _pool_call():
    mesh = plsc.VectorSubcoreMesh(core_axis_name="c", subcore_axis_name="s")
    return functools.partial(
        pl.kernel,
        out_type=jax.ShapeDtypeStruct((B * KG, 128), jnp.float32),
        mesh=mesh,
        scratch_types=[
            pltpu.VMEM((2, N), jnp.float32),
            pltpu.VMEM((96,), jnp.float32),
            pltpu.VMEM((2, KN), jnp.int32),
            pltpu.VMEM((2, KN, 128), jnp.float32),
            pltpu.VMEM((QPW, 128), jnp.float32),
            pltpu.SemaphoreType.DMA,
            pltpu.SemaphoreType.DMA,
            pltpu.SemaphoreType.DMA,
            pltpu.SemaphoreType.DMA,
        ],
    )(_pool_body)


def kernel(input, W0, b0, W1, b1, W2, b2):
    x = input.astype(jnp.float32)
    pts_t = jnp.transpose(x, (2, 0, 1))
    sel = _FPS(pts_t)

    x2d = jnp.pad(x.reshape(NROWS, CIN), ((0, NPAD - NROWS), (0, 2)))
    w0p = jnp.pad(W0, ((0, 2), (0, 0)))
    h2 = _MLP(x2d, w0p, b0[None], W1, b1[None], W2, b2[None])

    sel_bq = jnp.transpose(sel, (1, 0, 2))
    pts_bt = jnp.transpose(x, (0, 2, 1))
    d2 = _DIST(sel_bq, pts_bt).reshape(B * KG, N)
    pooled = _pool_call()(d2, h2)
    return jnp.concatenate(
        [sel_bq[:, :, :3], pooled.reshape(B, KG, 128)], axis=2)


# speedup vs baseline: 1.0588x; 1.0588x over previous
import functools

import numpy as np

import jax
import jax.numpy as jnp
from jax import lax
from jax.experimental import pallas as pl
from jax.experimental.pallas import tpu as pltpu
from jax.experimental.pallas import tpu_sc as plsc

B = 8
N = 4096
CIN = 6
KG = 512
KN = 64
R2 = np.float32(0.8 * 0.8)
NROWS = B * N
ZROW = NROWS
NPAD = NROWS + KG
NW = 32
QPW = (B * KG) // NW


def _fps_body(pts_ref, out_ref):
    lane = lax.broadcasted_iota(jnp.int32, (B, N), 1)

    def step(k, carry):
        min_d, last = carry
        m = lane == last
        qs = []
        for c in range(CIN):
            pc = pts_ref[c]
            qs.append(jnp.sum(jnp.where(m, pc, 0.0), axis=1, keepdims=True))
        row = jnp.concatenate(qs, axis=1)
        out_ref[pl.ds(k, 1)] = row[None]
        e = []
        for c in range(CIN):
            t = pts_ref[c] - qs[c]
            e.append(t * t)
        d = ((e[0] + e[4]) + e[2]) + ((e[1] + e[5]) + e[3])
        min_d = jnp.minimum(min_d, d)
        mx = jnp.max(min_d, axis=1, keepdims=True)
        nxt = jnp.min(jnp.where(min_d == mx, lane, jnp.int32(N)),
                      axis=1, keepdims=True)
        return min_d, nxt

    init = (jnp.full((B, N), jnp.inf, jnp.float32),
            jnp.zeros((B, 1), jnp.int32))
    lax.fori_loop(0, KG, step, init)


_FPS = pl.pallas_call(
    _fps_body,
    out_shape=jax.ShapeDtypeStruct((KG, B, CIN), jnp.float32),
)


_MLP_BLK = NPAD // 8


def _mlp_body(x_ref, w0_ref, b0_ref, w1_ref, b1_ref, w2_ref, b2_ref, o_ref):
    h = jnp.dot(x_ref[...], w0_ref[...], preferred_element_type=jnp.float32)
    h = jnp.maximum(h + b0_ref[...], 0.0)
    h = jnp.dot(h, w1_ref[...], preferred_element_type=jnp.float32)
    h = jnp.maximum(h + b1_ref[...], 0.0)
    h = jnp.dot(h, w2_ref[...], preferred_element_type=jnp.float32)
    o_ref[...] = jnp.maximum(h + b2_ref[...], 0.0)


_MLP = pl.pallas_call(
    _mlp_body,
    grid=(8,),
    in_specs=[
        pl.BlockSpec((_MLP_BLK, 8), lambda i: (i, 0)),
        pl.BlockSpec((8, 64), lambda i: (0, 0)),
        pl.BlockSpec((1, 64), lambda i: (0, 0)),
        pl.BlockSpec((64, 64), lambda i: (0, 0)),
        pl.BlockSpec((1, 64), lambda i: (0, 0)),
        pl.BlockSpec((64, 128), lambda i: (0, 0)),
        pl.BlockSpec((1, 128), lambda i: (0, 0)),
    ],
    out_specs=pl.BlockSpec((_MLP_BLK, 128), lambda i: (i, 0)),
    out_shape=jax.ShapeDtypeStruct((NPAD, 128), jnp.float32),
)


def _dist_body(p1_ref, pts_ref, o_ref):
    q = p1_ref[0]
    kt = pts_ref[0]
    n1 = jnp.sum(q * q, axis=1, keepdims=True)
    n2 = jnp.sum(kt * kt, axis=0, keepdims=True)
    dt = jnp.dot(q, kt, preferred_element_type=jnp.float32)
    o_ref[0] = (n1 + n2) - 2.0 * dt


_DIST = pl.pallas_call(
    _dist_body,
    grid=(8,),
    in_specs=[
        pl.BlockSpec((1, KG, CIN), lambda i: (i, 0, 0)),
        pl.BlockSpec((1, CIN, N), lambda i: (i, 0, 0)),
    ],
    out_specs=pl.BlockSpec((1, KG, N), lambda i: (i, 0, 0)),
    out_shape=jax.ShapeDtypeStruct((B, KG, N), jnp.float32),
)


_GV = 8
_NG = N // (16 * _GV)
_TRASH = 224
_IDXB = _TRASH + 16


def _pool_body(d2_hbm, h2_hbm, out_hbm,
               d2row_v, idxb_v, idx_v, rows_v, out_v,
               dsem_a, dsem_b, gsem_a, gsem_b):
    c = lax.axis_index("c")
    s = lax.axis_index("s")
    w = s * 2 + c
    b = w // 4
    qbase = w * QPW
    boff_f = (b * N).astype(jnp.float32)

    lane0 = lax.iota(jnp.int32, 16) == 0
    iota_f = lax.iota(jnp.int32, 16).astype(jnp.float32)
    zrow_f = np.float32(float(ZROW))
    zfill = jnp.full((16,), zrow_f, jnp.float32)

    def scan(dbuf):
        for u in range(6):
            idxb_v[pl.ds(u * 16, 16)] = zfill

        def gbody(g, cnt):
            def active():
                cc = cnt
                for v in range(_GV):
                    kv16 = (g * _GV + v) * 16
                    dd = d2row_v[dbuf, pl.ds(kv16, 16)]
                    mf = jnp.where(dd < R2, 1.0, 0.0).astype(jnp.float32)
                    vals_f = boff_f + kv16.astype(jnp.float32) + iota_f
                    for l in range(16):
                        keep_f = mf[l]
                        dst = jnp.where(keep_f > np.float32(0.5), cc,
                                        jnp.int32(_TRASH))
                        idxb_v[pl.ds(dst, 16)] = jnp.where(
                            lane0, vals_f[l], zrow_f)
                        cc = cc + keep_f.astype(jnp.int32)
                return cc

            return lax.cond(cnt < KN, active, lambda: cnt)

        lax.fori_loop(0, _NG, gbody, jnp.int32(0))

    def conv_idx(p):
        for u in range(4):
            idx_v[p, pl.ds(u * 16, 16)] = (
                idxb_v[pl.ds(u * 16, 16)].astype(jnp.int32))

    def maxq(p, qi):
        def rbody(r4, accs):
            for dr in range(4):
                r = r4 * 4 + dr
                accs = tuple(
                    jnp.maximum(accs[v], rows_v[p, r, pl.ds(v * 16, 16)])
                    for v in range(8))
            return accs

        accs = tuple(jnp.zeros((16,), jnp.float32) for _ in range(8))
        accs = lax.fori_loop(0, KN // 4, rbody, accs)
        for v in range(8):
            out_v[qi, pl.ds(v * 16, 16)] = accs[v]

    pltpu.async_copy(d2_hbm.at[qbase], d2row_v.at[0], dsem_a)

    def pair(j, carry):
        e = 2 * j
        o = e + 1
        pltpu.make_async_copy(
            d2_hbm.at[qbase + e], d2row_v.at[0], dsem_a).wait()
        pltpu.async_copy(d2_hbm.at[qbase + o], d2row_v.at[1], dsem_b)
        scan(0)
        conv_idx(0)
        pltpu.async_copy(h2_hbm.at[idx_v.at[0]], rows_v.at[0], gsem_a)

        @pl.when(j > 0)
        def _():
            pltpu.make_async_copy(
                h2_hbm.at[idx_v.at[1]], rows_v.at[1], gsem_b).wait()
            maxq(1, e - 1)

        pltpu.make_async_copy(
            d2_hbm.at[qbase + o], d2row_v.at[1], dsem_b).wait()

        @pl.when(j < QPW // 2 - 1)
        def _():
            pltpu.async_copy(
                d2_hbm.at[qbase + e + 2], d2row_v.at[0], dsem_a)

        scan(1)
        conv_idx(1)
        pltpu.async_copy(h2_hbm.at[idx_v.at[1]], rows_v.at[1], gsem_b)
        pltpu.make_async_copy(
            h2_hbm.at[idx_v.at[0]], rows_v.at[0], gsem_a).wait()
        maxq(0, e)
        return carry

    lax.fori_loop(0, QPW // 2, pair, 0)
    pltpu.make_async_copy(
        h2_hbm.at[idx_v.at[1]], rows_v.at[1], gsem_b).wait()
    maxq(1, QPW - 1)
    pltpu.sync_copy(out_v, out_hbm.at[pl.ds(qbase, QPW)])


@functools.cache
def _pool_call():
    mesh = plsc.VectorSubcoreMesh(core_axis_name="c", subcore_axis_name="s")
    return functools.partial(
        pl.kernel,
        out_type=jax.ShapeDtypeStruct((B * KG, 128), jnp.float32),
        mesh=mesh,
        scratch_types=[
            pltpu.VMEM((2, N), jnp.float32),
            pltpu.VMEM((_IDXB,), jnp.float32),
            pltpu.VMEM((2, KN), jnp.int32),
            pltpu.VMEM((2, KN, 128), jnp.float32),
            pltpu.VMEM((QPW, 128), jnp.float32),
            pltpu.SemaphoreType.DMA,
            pltpu.SemaphoreType.DMA,
            pltpu.SemaphoreType.DMA,
            pltpu.SemaphoreType.DMA,
        ],
    )(_pool_body)


def kernel(input, W0, b0, W1, b1, W2, b2):
    x = input.astype(jnp.float32)
    pts_t = jnp.transpose(x, (2, 0, 1))
    sel = _FPS(pts_t)

    x2d = jnp.pad(x.reshape(NROWS, CIN), ((0, NPAD - NROWS), (0, 2)))
    w0p = jnp.pad(W0, ((0, 2), (0, 0)))
    h2 = _MLP(x2d, w0p, b0[None], W1, b1[None], W2, b2[None])

    sel_bq = jnp.transpose(sel, (1, 0, 2))
    pts_bt = jnp.transpose(x, (0, 2, 1))
    d2 = _DIST(sel_bq, pts_bt).reshape(B * KG, N)
    pooled = _pool_call()(d2, h2)
    return jnp.concatenate(
        [sel_bq[:, :, :3], pooled.reshape(B, KG, 128)], axis=2)
